# Initial kernel scaffold; baseline (speedup 1.0000x reference)
import jax
import jax.numpy as jnp
from jax.experimental import pallas as pl


def kernel(x, pos, batch, W, b, gamma, beta):
    raise NotImplementedError("write your pallas kernel here")


# speedup vs baseline: 3.6885x; 3.6885x over previous
import functools

import jax
import jax.numpy as jnp
from jax import lax
from jax.experimental import pallas as pl
from jax.experimental.pallas import tpu as pltpu
from jax.experimental.pallas import tpu_sc as plsc

N = 10000
DIN = 512
F = 20
FP = 32
K = 16

KNN_BLOCK = 200

SC_NW = 32
SC_QPC = 8
SC_EPC = SC_QPC * K
SC_NCHUNK = N // SC_QPC
SC_TMAX = (SC_NCHUNK + SC_NW - 1) // SC_NW


def _mlp_body(x_ref, w_ref, b_ref, gamma_ref, beta_ref, noise_ref,
              emb_ref, embn_ref):
    h = jnp.dot(x_ref[...], w_ref[...], preferred_element_type=jnp.float32)
    h = h + b_ref[...]
    mean = jnp.mean(h, axis=0, keepdims=True)
    var = jnp.mean((h - mean) ** 2, axis=0, keepdims=True)
    h = (h - mean) / jnp.sqrt(var + 1e-5) * gamma_ref[...] + beta_ref[...]
    e = jnp.maximum(h, 0.0)
    n = x_ref.shape[0]
    z = jnp.zeros((n, FP - F), jnp.float32)
    emb_ref[...] = jnp.concatenate([e, z], axis=1)
    embn_ref[...] = e + noise_ref[...]


def _mlp(x, w, b, gamma, beta, noise):
    n = x.shape[0]
    out_shape = (jax.ShapeDtypeStruct((n, FP), jnp.float32),
                 jax.ShapeDtypeStruct((n, F), jnp.float32))
    return pl.pallas_call(_mlp_body, out_shape=out_shape)(
        x, w, b.reshape(1, F), gamma.reshape(1, F), beta.reshape(1, F), noise)


def _knn_body(feat_ref, q_ref, out_ref):
    n = feat_ref.shape[0]
    nrows = q_ref.shape[0]
    qc = q_ref[...]
    feat = feat_ref[...]
    sqf = jnp.sum(feat * feat, axis=1)
    sqq = jnp.sum(qc * qc, axis=1)
    g = lax.dot_general(qc, feat, (((1,), (1,)), ((), ())),
                        preferred_element_type=jnp.float32)
    d = sqq[:, None] - 2.0 * g + sqf[None, :]
    col = lax.broadcasted_iota(jnp.int32, (nrows, n), 1)
    rows = (pl.program_id(0) * nrows
            + lax.broadcasted_iota(jnp.int32, (nrows, 1), 0))
    inf = jnp.float32(jnp.inf)
    d = jnp.where(col == rows, inf, d)
    kiota = lax.broadcasted_iota(jnp.int32, (nrows, K), 1)
    out0 = jnp.zeros((nrows, K), jnp.int32)

    def round_(t, carry):
        d, out = carry
        m = jnp.min(d, axis=1, keepdims=True)
        j = jnp.min(jnp.where(d == m, col, n), axis=1, keepdims=True)
        out = jnp.where(kiota == t, j, out)
        d = jnp.where(col == j, inf, d)
        return d, out

    _, out = lax.fori_loop(0, K, round_, (d, out0))
    out_ref[...] = out


def _knn(feat):
    n, dp = feat.shape
    grid = n // KNN_BLOCK
    return pl.pallas_call(
        _knn_body,
        grid=(grid,),
        in_specs=[
            pl.BlockSpec((n, dp), lambda i: (0, 0)),
            pl.BlockSpec((KNN_BLOCK, dp), lambda i: (i, 0)),
        ],
        out_specs=pl.BlockSpec((KNN_BLOCK, K), lambda i: (i, 0)),
        out_shape=jax.ShapeDtypeStruct((n, K), jnp.int32),
    )(feat, feat)


def _score_body(embn_hbm, idx_hbm, pat_hbm, p_hbm,
                idx_v, pat_v, idxm_v, tgtm_v, srct_v, tgtt_v, p_v, sem):
    wid = lax.axis_index("s") * 2 + lax.axis_index("c")
    pltpu.sync_copy(pat_hbm, pat_v)

    def chunk(t, carry):
        c = t * SC_NW + wid

        @pl.when(c < SC_NCHUNK)
        def _():
            ebase = c * SC_EPC
            qbase = c * SC_QPC
            pltpu.sync_copy(idx_hbm.at[pl.ds(ebase, SC_EPC)], idx_v)
            for g in range(SC_QPC):
                gs = pl.ds(g * 16, 16)
                base = idx_v[gs] * F
                tq = jnp.full((16,), qbase * F, jnp.int32)
                for d in range(F):
                    idxm_v[d, gs] = base + d
                    tgtm_v[d, gs] = pat_v[d, gs] + tq

            def fire(d, cc):
                pltpu.async_copy(embn_hbm.at[idxm_v.at[d]], srct_v.at[d], sem)
                pltpu.async_copy(embn_hbm.at[tgtm_v.at[d]], tgtt_v.at[d], sem)
                return cc

            lax.fori_loop(0, F, fire, 0)

            def drain(d, cc):
                pltpu.make_async_copy(
                    embn_hbm.at[idxm_v.at[d]], srct_v.at[d], sem).wait()
                pltpu.make_async_copy(
                    embn_hbm.at[tgtm_v.at[d]], tgtt_v.at[d], sem).wait()
                return cc

            lax.fori_loop(0, F, drain, 0)
            for g in range(SC_QPC):
                gs = pl.ds(g * 16, 16)
                s0 = jnp.zeros((16,), jnp.float32)
                s1 = jnp.zeros((16,), jnp.float32)
                for d in range(0, F, 2):
                    f0 = srct_v[d, gs] - tgtt_v[d, gs]
                    f1 = srct_v[d + 1, gs] - tgtt_v[d + 1, gs]
                    s0 = s0 + f0 * f0
                    s1 = s1 + f1 * f1
                p_v[gs] = s0 + s1
            pltpu.sync_copy(p_v, p_hbm.at[pl.ds(ebase, SC_EPC)])

        return carry

    lax.fori_loop(0, SC_TMAX, chunk, 0)


def _score(embn_flat, idx, pat):
    mesh = plsc.VectorSubcoreMesh(core_axis_name="c", subcore_axis_name="s")
    kern = pl.kernel(
        _score_body,
        out_type=jax.ShapeDtypeStruct((N * K,), jnp.float32),
        mesh=mesh,
        scratch_types=[
            pltpu.VMEM((SC_EPC,), jnp.int32),
            pltpu.VMEM((F, SC_EPC), jnp.int32),
            pltpu.VMEM((F, SC_EPC), jnp.int32),
            pltpu.VMEM((F, SC_EPC), jnp.int32),
            pltpu.VMEM((F, SC_EPC), jnp.float32),
            pltpu.VMEM((F, SC_EPC), jnp.float32),
            pltpu.VMEM((SC_EPC,), jnp.float32),
            pltpu.SemaphoreType.DMA,
        ],
    )
    return kern(embn_flat, idx, pat)


def _finish_body(s_ref, p_ref):
    p_ref[...] = jnp.exp(-jnp.sqrt(s_ref[...]))


def _finish(s2d):
    return pl.pallas_call(
        _finish_body,
        out_shape=jax.ShapeDtypeStruct(s2d.shape, jnp.float32),
    )(s2d)


def kernel(x, pos, batch, W, b, gamma, beta):
    noise = jax.random.uniform(jax.random.key(42), (N, F), jnp.float32) * 1e-4
    emb, embn = _mlp(x, W, b, gamma, beta, noise)
    idx_emb = _knn(emb)
    idx_pos = _knn(pos)
    src = idx_emb.reshape(N * K)
    tgt = jnp.repeat(jnp.arange(N, dtype=jnp.int32), K)
    pat = ((jnp.arange(SC_EPC, dtype=jnp.int32)[None, :] // K) * F
           + jnp.arange(F, dtype=jnp.int32)[:, None])
    s = _score(embn.reshape(N * F), src, pat)
    p = _finish(s.reshape(N * K // 128, 128)).reshape(N * K)
    edges_large = jnp.stack([src, tgt], axis=0)
    soft_index_v = jnp.stack([p, tgt.astype(jnp.float32)], axis=0)
    pos_edges = jnp.stack([idx_pos.reshape(N * K), tgt], axis=0)
    edge_index = jnp.concatenate([edges_large, pos_edges], axis=1)
    return edges_large, soft_index_v, edge_index


# speedup vs baseline: 4.0759x; 1.1050x over previous
import functools

import jax
import jax.numpy as jnp
from jax import lax
from jax.experimental import pallas as pl
from jax.experimental.pallas import tpu as pltpu
from jax.experimental.pallas import tpu_sc as plsc

N = 10000
DIN = 512
F = 20
FP = 32
K = 16

KNN_BLOCK = 200

SC_NW = 32
SC_QPC = 8
SC_EPC = SC_QPC * K
SC_NCHUNK = N // SC_QPC
SC_TMAX = (SC_NCHUNK + SC_NW - 1) // SC_NW


def _mlp_body(x_ref, w_ref, b_ref, gamma_ref, beta_ref, noise_ref,
              emb_ref, embn_ref):
    h = jnp.dot(x_ref[...], w_ref[...], preferred_element_type=jnp.float32)
    h = h + b_ref[...]
    mean = jnp.mean(h, axis=0, keepdims=True)
    var = jnp.mean((h - mean) ** 2, axis=0, keepdims=True)
    h = (h - mean) / jnp.sqrt(var + 1e-5) * gamma_ref[...] + beta_ref[...]
    e = jnp.maximum(h, 0.0)
    n = x_ref.shape[0]
    z = jnp.zeros((n, FP - F), jnp.float32)
    emb_ref[...] = jnp.concatenate([e, z], axis=1)
    embn_ref[...] = e + noise_ref[...]


def _mlp(x, w, b, gamma, beta, noise):
    n = x.shape[0]
    out_shape = (jax.ShapeDtypeStruct((n, FP), jnp.float32),
                 jax.ShapeDtypeStruct((n, F), jnp.float32))
    return pl.pallas_call(_mlp_body, out_shape=out_shape)(
        x, w, b.reshape(1, F), gamma.reshape(1, F), beta.reshape(1, F), noise)


def _knn_body(feat_ref, q_ref, out_ref):
    n = feat_ref.shape[0]
    nrows = q_ref.shape[0]
    qc = q_ref[...]
    feat = feat_ref[...]
    sqf = jnp.sum(feat * feat, axis=1)
    sqq = jnp.sum(qc * qc, axis=1)
    g = lax.dot_general(qc, feat, (((1,), (1,)), ((), ())),
                        preferred_element_type=jnp.float32)
    d = sqq[:, None] - 2.0 * g + sqf[None, :]
    col = lax.broadcasted_iota(jnp.int32, (nrows, n), 1)
    rows = (pl.program_id(0) * nrows
            + lax.broadcasted_iota(jnp.int32, (nrows, 1), 0))
    inf = jnp.float32(jnp.inf)
    kiota = lax.broadcasted_iota(jnp.int32, (nrows, K), 1)
    out0 = jnp.zeros((nrows, K), jnp.int32)

    def round_(t, carry):
        d, out, jprev = carry
        dm = jnp.where(col == jprev, inf, d)
        m = jnp.min(dm, axis=1, keepdims=True)
        j = jnp.min(jnp.where(dm == m, col, n), axis=1, keepdims=True)
        out = jnp.where(kiota == t, j, out)
        return dm, out, j

    _, out, _ = lax.fori_loop(0, K, round_, (d, out0, rows))
    out_ref[...] = out


def _knn(feat):
    n, dp = feat.shape
    grid = n // KNN_BLOCK
    return pl.pallas_call(
        _knn_body,
        grid=(grid,),
        in_specs=[
            pl.BlockSpec((n, dp), lambda i: (0, 0)),
            pl.BlockSpec((KNN_BLOCK, dp), lambda i: (i, 0)),
        ],
        out_specs=pl.BlockSpec((KNN_BLOCK, K), lambda i: (i, 0)),
        out_shape=jax.ShapeDtypeStruct((n, K), jnp.int32),
    )(feat, feat)


def _score_body(embn_hbm, idx_hbm, pat_hbm, p_hbm,
                pat_v,
                idx_v0, idxm_v0, tgtm_v0, srct_v0, tgtt_v0, p_v0, sem0,
                idx_v1, idxm_v1, tgtm_v1, srct_v1, tgtt_v1, p_v1, sem1):
    wid = lax.axis_index("s") * 2 + lax.axis_index("c")
    pltpu.sync_copy(pat_hbm, pat_v)
    bufs = ((idx_v0, idxm_v0, tgtm_v0, srct_v0, tgtt_v0, p_v0, sem0),
            (idx_v1, idxm_v1, tgtm_v1, srct_v1, tgtt_v1, p_v1, sem1))

    def build_fire(c, buf):
        idx_v, idxm_v, tgtm_v, srct_v, tgtt_v, p_v, sem = buf

        @pl.when(c < SC_NCHUNK)
        def _():
            ebase = c * SC_EPC
            qbase = c * SC_QPC
            pltpu.sync_copy(idx_hbm.at[pl.ds(ebase, SC_EPC)], idx_v)
            for g in range(SC_QPC):
                gs = pl.ds(g * 16, 16)
                base = idx_v[gs] * F
                tq = jnp.full((16,), qbase * F, jnp.int32)
                for d in range(F):
                    idxm_v[d, gs] = base + d
                    tgtm_v[d, gs] = pat_v[d, gs] + tq

            def fire(d, cc):
                pltpu.async_copy(embn_hbm.at[idxm_v.at[d]], srct_v.at[d], sem)
                pltpu.async_copy(embn_hbm.at[tgtm_v.at[d]], tgtt_v.at[d], sem)
                return cc

            lax.fori_loop(0, F, fire, 0)

    def drain_compute(c, buf):
        idx_v, idxm_v, tgtm_v, srct_v, tgtt_v, p_v, sem = buf

        @pl.when(c < SC_NCHUNK)
        def _():
            ebase = c * SC_EPC

            def drain(d, cc):
                pltpu.make_async_copy(
                    embn_hbm.at[idxm_v.at[d]], srct_v.at[d], sem).wait()
                pltpu.make_async_copy(
                    embn_hbm.at[tgtm_v.at[d]], tgtt_v.at[d], sem).wait()
                return cc

            lax.fori_loop(0, F, drain, 0)
            for g in range(SC_QPC):
                gs = pl.ds(g * 16, 16)
                s0 = jnp.zeros((16,), jnp.float32)
                s1 = jnp.zeros((16,), jnp.float32)
                for d in range(0, F, 2):
                    f0 = srct_v[d, gs] - tgtt_v[d, gs]
                    f1 = srct_v[d + 1, gs] - tgtt_v[d + 1, gs]
                    s0 = s0 + f0 * f0
                    s1 = s1 + f1 * f1
                p_v[gs] = s0 + s1
            pltpu.sync_copy(p_v, p_hbm.at[pl.ds(ebase, SC_EPC)])

    build_fire(wid, bufs[0])

    def pair(u, carry):
        t0 = 2 * u
        build_fire((t0 + 1) * SC_NW + wid, bufs[1])
        drain_compute(t0 * SC_NW + wid, bufs[0])
        build_fire((t0 + 2) * SC_NW + wid, bufs[0])
        drain_compute((t0 + 1) * SC_NW + wid, bufs[1])
        return carry

    lax.fori_loop(0, SC_TMAX // 2, pair, 0)


def _score(embn_flat, idx, pat):
    mesh = plsc.VectorSubcoreMesh(core_axis_name="c", subcore_axis_name="s")
    buf_scratch = [
        pltpu.VMEM((SC_EPC,), jnp.int32),
        pltpu.VMEM((F, SC_EPC), jnp.int32),
        pltpu.VMEM((F, SC_EPC), jnp.int32),
        pltpu.VMEM((F, SC_EPC), jnp.float32),
        pltpu.VMEM((F, SC_EPC), jnp.float32),
        pltpu.VMEM((SC_EPC,), jnp.float32),
        pltpu.SemaphoreType.DMA,
    ]
    kern = pl.kernel(
        _score_body,
        out_type=jax.ShapeDtypeStruct((N * K,), jnp.float32),
        mesh=mesh,
        scratch_types=[pltpu.VMEM((F, SC_EPC), jnp.int32)]
        + buf_scratch + buf_scratch,
    )
    return kern(embn_flat, idx, pat)


def _finish_body(s_ref, p_ref):
    p_ref[...] = jnp.exp(-jnp.sqrt(s_ref[...]))


def _finish(s2d):
    return pl.pallas_call(
        _finish_body,
        out_shape=jax.ShapeDtypeStruct(s2d.shape, jnp.float32),
    )(s2d)


def kernel(x, pos, batch, W, b, gamma, beta):
    noise = jax.random.uniform(jax.random.key(42), (N, F), jnp.float32) * 1e-4
    emb, embn = _mlp(x, W, b, gamma, beta, noise)
    idx_emb = _knn(emb)
    idx_pos = _knn(pos)
    src = idx_emb.reshape(N * K)
    tgt = jnp.repeat(jnp.arange(N, dtype=jnp.int32), K)
    pat = ((jnp.arange(SC_EPC, dtype=jnp.int32)[None, :] // K) * F
           + jnp.arange(F, dtype=jnp.int32)[:, None])
    s = _score(embn.reshape(N * F), src, pat)
    p = _finish(s.reshape(N * K // 128, 128)).reshape(N * K)
    edges_large = jnp.stack([src, tgt], axis=0)
    soft_index_v = jnp.stack([p, tgt.astype(jnp.float32)], axis=0)
    pos_edges = jnp.stack([idx_pos.reshape(N * K), tgt], axis=0)
    edge_index = jnp.concatenate([edges_large, pos_edges], axis=1)
    return edges_large, soft_index_v, edge_index


# speedup vs baseline: 4.0804x; 1.0011x over previous
import functools

import jax
import jax.numpy as jnp
from jax import lax
from jax.experimental import pallas as pl
from jax.experimental.pallas import tpu as pltpu
from jax.experimental.pallas import tpu_sc as plsc

N = 10000
DIN = 512
F = 20
FP = 32
K = 16

KNN_BLOCK = 200

SC_NW = 32
SC_QPC = 8
SC_EPC = SC_QPC * K
SC_NCHUNK = N // SC_QPC
SC_TMAX = (SC_NCHUNK + SC_NW - 1) // SC_NW


def _mlp_body(x_ref, w_ref, b_ref, gamma_ref, beta_ref, noise_ref,
              emb_ref, embn_ref):
    h = jnp.dot(x_ref[...], w_ref[...], preferred_element_type=jnp.float32)
    h = h + b_ref[...]
    mean = jnp.mean(h, axis=0, keepdims=True)
    var = jnp.mean((h - mean) ** 2, axis=0, keepdims=True)
    h = (h - mean) / jnp.sqrt(var + 1e-5) * gamma_ref[...] + beta_ref[...]
    e = jnp.maximum(h, 0.0)
    n = x_ref.shape[0]
    z = jnp.zeros((n, FP - F), jnp.float32)
    emb_ref[...] = jnp.concatenate([e, z], axis=1)
    embn_ref[...] = jnp.concatenate([e + noise_ref[...], z], axis=1)


def _mlp(x, w, b, gamma, beta, noise):
    n = x.shape[0]
    out_shape = (jax.ShapeDtypeStruct((n, FP), jnp.float32),
                 jax.ShapeDtypeStruct((n, FP), jnp.float32))
    return pl.pallas_call(_mlp_body, out_shape=out_shape)(
        x, w, b.reshape(1, F), gamma.reshape(1, F), beta.reshape(1, F), noise)


def _knn_body(feat_ref, q_ref, out_ref):
    n = feat_ref.shape[0]
    nrows = q_ref.shape[0]
    qc = q_ref[...]
    feat = feat_ref[...]
    sqf = jnp.sum(feat * feat, axis=1)
    sqq = jnp.sum(qc * qc, axis=1)
    g = lax.dot_general(qc, feat, (((1,), (1,)), ((), ())),
                        preferred_element_type=jnp.float32)
    d = sqq[:, None] - 2.0 * g + sqf[None, :]
    col = lax.broadcasted_iota(jnp.int32, (nrows, n), 1)
    rows = (pl.program_id(0) * nrows
            + lax.broadcasted_iota(jnp.int32, (nrows, 1), 0))
    inf = jnp.float32(jnp.inf)
    kiota = lax.broadcasted_iota(jnp.int32, (nrows, K), 1)
    out0 = jnp.zeros((nrows, K), jnp.int32)

    def round_(t, carry):
        d, out, jprev = carry
        dm = jnp.where(col == jprev, inf, d)
        m = jnp.min(dm, axis=1, keepdims=True)
        j = jnp.min(jnp.where(dm == m, col, n), axis=1, keepdims=True)
        out = jnp.where(kiota == t, j, out)
        return dm, out, j

    _, out, _ = lax.fori_loop(0, K, round_, (d, out0, rows))
    out_ref[...] = out


def _knn(feat):
    n, dp = feat.shape
    grid = n // KNN_BLOCK
    return pl.pallas_call(
        _knn_body,
        grid=(grid,),
        in_specs=[
            pl.BlockSpec((n, dp), lambda i: (0, 0)),
            pl.BlockSpec((KNN_BLOCK, dp), lambda i: (i, 0)),
        ],
        out_specs=pl.BlockSpec((KNN_BLOCK, K), lambda i: (i, 0)),
        out_shape=jax.ShapeDtypeStruct((n, K), jnp.int32),
    )(feat, feat)


def _score_body(embn_hbm, idx_hbm, p_hbm,
                idx_v0, row_v0, tgt_v0, p_v0, sem0,
                idx_v1, row_v1, tgt_v1, p_v1, sem1):
    wid = lax.axis_index("s") * 2 + lax.axis_index("c")
    lane = lax.iota(jnp.int32, 16)
    bufs = ((idx_v0, row_v0, tgt_v0, p_v0, sem0),
            (idx_v1, row_v1, tgt_v1, p_v1, sem1))

    def build_fire(c, buf):
        idx_v, row_v, tgt_v, p_v, sem = buf

        @pl.when(c < SC_NCHUNK)
        def _():
            ebase = c * SC_EPC
            qbase = c * SC_QPC
            pltpu.sync_copy(idx_hbm.at[pl.ds(ebase, SC_EPC)], idx_v)
            pltpu.async_copy(embn_hbm.at[idx_v], row_v, sem)
            pltpu.sync_copy(embn_hbm.at[pl.ds(qbase, SC_QPC)], tgt_v)

    def drain_compute(c, buf):
        idx_v, row_v, tgt_v, p_v, sem = buf

        @pl.when(c < SC_NCHUNK)
        def _():
            ebase = c * SC_EPC
            pltpu.make_async_copy(embn_hbm.at[idx_v], row_v, sem).wait()
            for g in range(SC_QPC):
                b0 = tgt_v[g, pl.ds(0, 16)]
                b1 = tgt_v[g, pl.ds(16, 16)]
                acc = jnp.zeros((16,), jnp.float32)
                for e16 in range(16):
                    e = g * 16 + e16
                    f0 = row_v[e, pl.ds(0, 16)] - b0
                    f1 = row_v[e, pl.ds(16, 16)] - b1
                    pv = f0 * f0 + f1 * f1
                    pv = pv + jnp.take(pv, lane ^ 8)
                    pv = pv + jnp.take(pv, lane ^ 4)
                    pv = pv + jnp.take(pv, lane ^ 2)
                    pv = pv + jnp.take(pv, lane ^ 1)
                    acc = jnp.where(lane == e16, pv, acc)
                p_v[pl.ds(g * 16, 16)] = acc
            pltpu.sync_copy(p_v, p_hbm.at[pl.ds(ebase, SC_EPC)])

    build_fire(wid, bufs[0])

    def pair(u, carry):
        t0 = 2 * u
        build_fire((t0 + 1) * SC_NW + wid, bufs[1])
        drain_compute(t0 * SC_NW + wid, bufs[0])
        build_fire((t0 + 2) * SC_NW + wid, bufs[0])
        drain_compute((t0 + 1) * SC_NW + wid, bufs[1])
        return carry

    lax.fori_loop(0, SC_TMAX // 2, pair, 0)


def _score(embn, idx):
    mesh = plsc.VectorSubcoreMesh(core_axis_name="c", subcore_axis_name="s")
    buf_scratch = [
        pltpu.VMEM((SC_EPC,), jnp.int32),
        pltpu.VMEM((SC_EPC, FP), jnp.float32),
        pltpu.VMEM((SC_QPC, FP), jnp.float32),
        pltpu.VMEM((SC_EPC,), jnp.float32),
        pltpu.SemaphoreType.DMA,
    ]
    kern = pl.kernel(
        _score_body,
        out_type=jax.ShapeDtypeStruct((N * K,), jnp.float32),
        mesh=mesh,
        compiler_params=pltpu.CompilerParams(use_tc_tiling_on_sc=False),
        scratch_types=buf_scratch + buf_scratch,
    )
    return kern(embn, idx)


def _finish_body(s_ref, p_ref):
    p_ref[...] = jnp.exp(-jnp.sqrt(s_ref[...]))


def _finish(s2d):
    return pl.pallas_call(
        _finish_body,
        out_shape=jax.ShapeDtypeStruct(s2d.shape, jnp.float32),
    )(s2d)


def kernel(x, pos, batch, W, b, gamma, beta):
    noise = jax.random.uniform(jax.random.key(42), (N, F), jnp.float32) * 1e-4
    emb, embn = _mlp(x, W, b, gamma, beta, noise)
    idx_emb = _knn(emb)
    idx_pos = _knn(pos)
    src = idx_emb.reshape(N * K)
    tgt = jnp.repeat(jnp.arange(N, dtype=jnp.int32), K)
    s = _score(embn, src)
    p = _finish(s.reshape(N * K // 128, 128)).reshape(N * K)
    edges_large = jnp.stack([src, tgt], axis=0)
    soft_index_v = jnp.stack([p, tgt.astype(jnp.float32)], axis=0)
    pos_edges = jnp.stack([idx_pos.reshape(N * K), tgt], axis=0)
    edge_index = jnp.concatenate([edges_large, pos_edges], axis=1)
    return edges_large, soft_index_v, edge_index


# speedup vs baseline: 5.5270x; 1.3545x over previous
import jax
import jax.numpy as jnp
from jax import lax
from jax.experimental import pallas as pl
from jax.experimental.pallas import tpu as pltpu
from jax.experimental.pallas import tpu_sc as plsc

N = 10000
DIN = 512
F = 20
FP = 32
K = 16

KNN_BLOCK = 200

SC_NW = 32
SC_QPC = 8
SC_EPC = SC_QPC * K
SC_NCHUNK = N // SC_QPC
SC_TMAX = (SC_NCHUNK + SC_NW - 1) // SC_NW


def _mlp_body(x_ref, w_ref, b_ref, gamma_ref, beta_ref, noise_ref,
              emb_ref, embn_ref):
    h = jnp.dot(x_ref[...], w_ref[...], preferred_element_type=jnp.float32)
    h = h + b_ref[...]
    mean = jnp.mean(h, axis=0, keepdims=True)
    var = jnp.mean((h - mean) ** 2, axis=0, keepdims=True)
    h = (h - mean) / jnp.sqrt(var + 1e-5) * gamma_ref[...] + beta_ref[...]
    e = jnp.maximum(h, 0.0)
    n = x_ref.shape[0]
    z = jnp.zeros((n, FP - F), jnp.float32)
    emb_ref[...] = jnp.concatenate([e, z], axis=1)
    embn_ref[...] = jnp.concatenate([e + noise_ref[...], z], axis=1)


def _mlp(x, w, b, gamma, beta, noise):
    n = x.shape[0]
    out_shape = (jax.ShapeDtypeStruct((n, FP), jnp.float32),
                 jax.ShapeDtypeStruct((n, FP), jnp.float32))
    return pl.pallas_call(_mlp_body, out_shape=out_shape)(
        x, w, b.reshape(1, F), gamma.reshape(1, F), beta.reshape(1, F), noise)


def _knn_body(feat_ref, q_ref, out_ref, d_ref):
    n = feat_ref.shape[0]
    nrows = q_ref.shape[0]
    qc = q_ref[...]
    feat = feat_ref[...]
    sqf = jnp.sum(feat * feat, axis=1)
    sqq = jnp.sum(qc * qc, axis=1)
    g = lax.dot_general(qc, feat, (((1,), (1,)), ((), ())),
                        preferred_element_type=jnp.float32)
    d_ref[...] = sqq[:, None] - 2.0 * g + sqf[None, :]
    col = lax.broadcasted_iota(jnp.int32, (nrows, n), 1)
    rows = (pl.program_id(0) * nrows
            + lax.broadcasted_iota(jnp.int32, (nrows, 1), 0))
    inf = jnp.float32(jnp.inf)
    kiota = lax.broadcasted_iota(jnp.int32, (nrows, K), 1)
    out0 = jnp.zeros((nrows, K), jnp.int32)

    def round_(t, carry):
        out, jprev = carry
        dm = jnp.where(col == jprev, inf, d_ref[...])
        d_ref[...] = dm
        m = jnp.min(dm, axis=1, keepdims=True)
        j = jnp.min(jnp.where(dm == m, col, n), axis=1, keepdims=True)
        out = jnp.where(kiota == t, j, out)
        return out, j

    out, _ = lax.fori_loop(0, K, round_, (out0, rows))
    out_ref[...] = out


def _knn(feat):
    n, dp = feat.shape
    grid = n // KNN_BLOCK
    return pl.pallas_call(
        _knn_body,
        grid=(grid,),
        in_specs=[
            pl.BlockSpec((n, dp), lambda i: (0, 0)),
            pl.BlockSpec((KNN_BLOCK, dp), lambda i: (i, 0)),
        ],
        out_specs=pl.BlockSpec((KNN_BLOCK, K), lambda i: (i, 0)),
        out_shape=jax.ShapeDtypeStruct((n, K), jnp.int32),
        scratch_shapes=[pltpu.VMEM((KNN_BLOCK, n), jnp.float32)],
    )(feat, feat)


def _score_body(embn_hbm, idx_hbm, p_hbm,
                idx_v0, row_v0, tgt_v0, p_v0, sem0,
                idx_v1, row_v1, tgt_v1, p_v1, sem1):
    wid = lax.axis_index("s") * 2 + lax.axis_index("c")
    lane = lax.iota(jnp.int32, 16)
    bufs = ((idx_v0, row_v0, tgt_v0, p_v0, sem0),
            (idx_v1, row_v1, tgt_v1, p_v1, sem1))

    def build_fire(c, buf):
        idx_v, row_v, tgt_v, p_v, sem = buf

        @pl.when(c < SC_NCHUNK)
        def _():
            ebase = c * SC_EPC
            qbase = c * SC_QPC
            pltpu.sync_copy(idx_hbm.at[pl.ds(ebase, SC_EPC)], idx_v)
            pltpu.async_copy(embn_hbm.at[idx_v], row_v, sem)
            pltpu.sync_copy(embn_hbm.at[pl.ds(qbase, SC_QPC)], tgt_v)

    def drain_compute(c, buf):
        idx_v, row_v, tgt_v, p_v, sem = buf

        @pl.when(c < SC_NCHUNK)
        def _():
            ebase = c * SC_EPC
            pltpu.make_async_copy(embn_hbm.at[idx_v], row_v, sem).wait()
            for g in range(SC_QPC):
                b0 = tgt_v[g, pl.ds(0, 16)]
                b1 = tgt_v[g, pl.ds(16, 16)]
                acc = jnp.zeros((16,), jnp.float32)
                for e16 in range(16):
                    e = g * 16 + e16
                    f0 = row_v[e, pl.ds(0, 16)] - b0
                    f1 = row_v[e, pl.ds(16, 16)] - b1
                    pv = f0 * f0 + f1 * f1
                    pv = pv + jnp.take(pv, lane ^ 8)
                    pv = pv + jnp.take(pv, lane ^ 4)
                    pv = pv + jnp.take(pv, lane ^ 2)
                    pv = pv + jnp.take(pv, lane ^ 1)
                    acc = jnp.where(lane == e16, pv, acc)
                p_v[pl.ds(g * 16, 16)] = acc
            pltpu.sync_copy(p_v, p_hbm.at[pl.ds(ebase, SC_EPC)])

    build_fire(wid, bufs[0])

    def pair(u, carry):
        t0 = 2 * u
        build_fire((t0 + 1) * SC_NW + wid, bufs[1])
        drain_compute(t0 * SC_NW + wid, bufs[0])
        build_fire((t0 + 2) * SC_NW + wid, bufs[0])
        drain_compute((t0 + 1) * SC_NW + wid, bufs[1])
        return carry

    lax.fori_loop(0, SC_TMAX // 2, pair, 0)


def _score(embn, idx):
    mesh = plsc.VectorSubcoreMesh(core_axis_name="c", subcore_axis_name="s")
    buf_scratch = [
        pltpu.VMEM((SC_EPC,), jnp.int32),
        pltpu.VMEM((SC_EPC, FP), jnp.float32),
        pltpu.VMEM((SC_QPC, FP), jnp.float32),
        pltpu.VMEM((SC_EPC,), jnp.float32),
        pltpu.SemaphoreType.DMA,
    ]
    kern = pl.kernel(
        _score_body,
        out_type=jax.ShapeDtypeStruct((N * K,), jnp.float32),
        mesh=mesh,
        compiler_params=pltpu.CompilerParams(use_tc_tiling_on_sc=False),
        scratch_types=buf_scratch + buf_scratch,
    )
    return kern(embn, idx)


def _finish_body(s_ref, p_ref):
    p_ref[...] = jnp.exp(-jnp.sqrt(s_ref[...]))


def _finish(s2d):
    return pl.pallas_call(
        _finish_body,
        out_shape=jax.ShapeDtypeStruct(s2d.shape, jnp.float32),
    )(s2d)


def kernel(x, pos, batch, W, b, gamma, beta):
    noise = jax.random.uniform(jax.random.key(42), (N, F), jnp.float32) * 1e-4
    emb, embn = _mlp(x, W, b, gamma, beta, noise)
    idx_emb = _knn(emb)
    idx_pos = _knn(pos)
    src = idx_emb.reshape(N * K)
    tgt = jnp.repeat(jnp.arange(N, dtype=jnp.int32), K)
    s = _score(embn, src)
    p = _finish(s.reshape(N * K // 128, 128)).reshape(N * K)
    edges_large = jnp.stack([src, tgt], axis=0)
    soft_index_v = jnp.stack([p, tgt.astype(jnp.float32)], axis=0)
    pos_edges = jnp.stack([idx_pos.reshape(N * K), tgt], axis=0)
    edge_index = jnp.concatenate([edges_large, pos_edges], axis=1)
    return edges_large, soft_index_v, edge_index
